# Initial kernel scaffold; baseline (speedup 1.0000x reference)
import jax
import jax.numpy as jnp
from jax.experimental import pallas as pl


def kernel(input, table):
    raise NotImplementedError("write your pallas kernel here")


# speedup vs baseline: 3.1879x; 3.1879x over previous
import functools

import jax
import jax.numpy as jnp
from jax import lax
from jax.experimental import pallas as pl
from jax.experimental.pallas import tpu as pltpu
from jax.experimental.pallas import tpu_sc as plsc

NTOKEN = 100000
D = 128
NC = 2
NS = 16
NW = NC * NS
CHUNK = 400


def _make_gather(B: int):
  b_per_w = B // NW
  n_steps = b_per_w // CHUNK
  mesh = plsc.VectorSubcoreMesh(
      core_axis_name="c", subcore_axis_name="s", num_cores=NC, num_subcores=NS
  )

  @functools.partial(
      pl.kernel,
      mesh=mesh,
      out_type=jax.ShapeDtypeStruct((B, D), jnp.float32),
      scratch_types=[
          pltpu.VMEM((CHUNK,), jnp.int32),
          pltpu.VMEM((CHUNK, D), jnp.float32),
          pltpu.SemaphoreType.DMA,
      ],
  )
  def gather_kernel(idx_hbm, table_hbm, out_hbm, idx_v, rows_v, sem):
    wid = lax.axis_index("s") * NC + lax.axis_index("c")
    base = wid * b_per_w

    def body(i, carry):
      off = base + i * CHUNK
      pltpu.sync_copy(idx_hbm.at[pl.ds(off, CHUNK)], idx_v)
      pltpu.async_copy(table_hbm.at[idx_v], rows_v, sem).wait()
      pltpu.sync_copy(rows_v, out_hbm.at[pl.ds(off, CHUNK)])
      return carry

    lax.fori_loop(0, n_steps, body, 0)

  return gather_kernel


@jax.jit
def kernel(input, table):
  shape = input.shape
  flat_idx = input.reshape(-1).astype(jnp.int32)
  out = _make_gather(flat_idx.shape[0])(flat_idx, table)
  return out.reshape(*shape, D)


# speedup vs baseline: 3.3478x; 1.0501x over previous
import functools

import jax
import jax.numpy as jnp
from jax import lax
from jax.experimental import pallas as pl
from jax.experimental.pallas import tpu as pltpu
from jax.experimental.pallas import tpu_sc as plsc

NTOKEN = 100000
D = 128
NC = 2
NS = 16
NW = NC * NS
CHUNK = 400
N_BUF = 2


def _make_gather(B: int):
  b_per_w = B // NW
  n_steps = b_per_w // CHUNK
  mesh = plsc.VectorSubcoreMesh(
      core_axis_name="c", subcore_axis_name="s", num_cores=NC, num_subcores=NS
  )

  @functools.partial(
      pl.kernel,
      mesh=mesh,
      out_type=jax.ShapeDtypeStruct((B, D), jnp.float32),
      scratch_types=[
          pltpu.VMEM((b_per_w,), jnp.int32),
          [pltpu.VMEM((CHUNK, D), jnp.float32) for _ in range(N_BUF)],
          [pltpu.SemaphoreType.DMA for _ in range(N_BUF)],
      ],
  )
  def gather_kernel(idx_hbm, table_hbm, out_hbm, idx_v, bufs, sems):
    wid = lax.axis_index("s") * NC + lax.axis_index("c")
    base = wid * b_per_w
    pltpu.sync_copy(idx_hbm.at[pl.ds(base, b_per_w)], idx_v)

    def start_gather(step, b):
      pltpu.async_copy(
          table_hbm.at[idx_v.at[pl.ds(step * CHUNK, CHUNK)]], bufs[b], sems[b]
      )

    for b in range(min(N_BUF, n_steps)):
      start_gather(b, b)
    for i in range(n_steps):
      b = i % N_BUF
      pltpu.make_async_copy(
          table_hbm.at[idx_v.at[pl.ds(i * CHUNK, CHUNK)]], bufs[b], sems[b]
      ).wait()
      pltpu.sync_copy(bufs[b], out_hbm.at[pl.ds(base + i * CHUNK, CHUNK)])
      if i + N_BUF < n_steps:
        start_gather(i + N_BUF, b)

  return gather_kernel


@jax.jit
def kernel(input, table):
  shape = input.shape
  flat_idx = input.reshape(-1).astype(jnp.int32)
  out = _make_gather(flat_idx.shape[0])(flat_idx, table)
  return out.reshape(*shape, D)
